# baseline (speedup 1.0000x reference)
import jax
import jax.numpy as jnp
from jax.experimental import pallas as pl
from jax.experimental.pallas import tpu as pltpu


def _maxpool_kernel(x_ref, o_ref):
    k, ho, w2 = x_ref.shape
    w = w2 // 2
    x = x_ref[...]
    a = jnp.maximum(x[:, :, 0:w], x[:, :, w:w2])
    m = jnp.maximum(a, pltpu.roll(a, w - 1, 2))
    lane = jax.lax.broadcasted_iota(jnp.int32, (k, ho, 128), 2)
    g0 = jnp.take_along_axis(m[:, :, 0:128], (2 * lane) & 127, axis=2)
    g1 = jnp.take_along_axis(m[:, :, 96:224], (2 * lane + 32) & 127, axis=2)
    out = jnp.where(lane < 64, g0, g1)
    o_ref[...] = out[:, :, 0 : w // 2]


def kernel(x):
    N, C, H, W = x.shape
    HO, WO = H // 2, W // 2
    NC = N * C
    K = 8
    xv = x.reshape(NC, HO, 2 * W)
    out = pl.pallas_call(
        _maxpool_kernel,
        grid=(NC // K,),
        in_specs=[pl.BlockSpec((K, HO, 2 * W), lambda i: (i, 0, 0))],
        out_specs=pl.BlockSpec((K, HO, WO), lambda i: (i, 0, 0)),
        out_shape=jax.ShapeDtypeStruct((NC, HO, WO), x.dtype),
        compiler_params=pltpu.CompilerParams(
            dimension_semantics=("parallel",),
        ),
    )(xv)
    return out.reshape(N, C, HO, WO)
